# baseline (speedup 1.0000x reference)
import jax
import jax.numpy as jnp
from jax.experimental import pallas as pl

_CBLK = 8


def _bcast_body(a_ref, b_ref, out_ref):
    plane = a_ref[...][:, :, None] + b_ref[...][:, None, :]
    out_ref[...] = jnp.broadcast_to(plane[None], out_ref.shape)


def kernel(mask, row_embed, col_embed):
    batch = mask.shape[0]
    h, w = mask.shape[-2], mask.shape[-1]
    nf = row_embed.shape[1]
    c_total = 2 * nf
    zeros = jnp.zeros((nf, h), dtype=row_embed.dtype)
    a_tab = jnp.concatenate([zeros, row_embed.T], axis=0)
    b_tab = jnp.concatenate([col_embed.T, zeros], axis=0)

    grid = (c_total // _CBLK,)
    out = pl.pallas_call(
        _bcast_body,
        grid=grid,
        in_specs=[
            pl.BlockSpec((_CBLK, h), lambda j: (j, 0)),
            pl.BlockSpec((_CBLK, w), lambda j: (j, 0)),
        ],
        out_specs=pl.BlockSpec((batch, _CBLK, h, w), lambda j: (0, j, 0, 0)),
        out_shape=jax.ShapeDtypeStruct((batch, c_total, h, w), row_embed.dtype),
    )(a_tab, b_tab)
    return out
